# baseline (speedup 1.0000x reference)
import functools

import jax
import jax.numpy as jnp
from jax import lax
from jax.experimental import pallas as pl
from jax.experimental.pallas import tpu as pltpu
from jax.experimental.pallas import tpu_sc as plsc

N = 10000
E = 320000
D = 128
NC = 2
NS = 16
NW = NC * NS
L = 16
CH = 80
NCH = (E // NW) // CH
EW = E // NW
NP = 10240
NPT = NP // NS
GRP = CH // L


def _ew_tc(ec_ref, ew_ref):
    e = ec_ref[...]
    mn = jnp.min(e)
    mx = jnp.max(e)
    ew_ref[...] = (e - mn) / (mx - mn)


def _edge_weights(ec2d):
    return pl.pallas_call(
        _ew_tc,
        out_shape=jax.ShapeDtypeStruct(ec2d.shape, jnp.float32),
    )(ec2d)


def _dis_tc(degp_ref, dis_ref):
    d = degp_ref[:1, :N] + degp_ref[1:, :N] + 1.0
    dis_ref[...] = jax.lax.rsqrt(jnp.maximum(d, 1e-12))


def _deg_inv_sqrt(degp2d):
    return pl.pallas_call(
        _dis_tc,
        out_shape=jax.ShapeDtypeStruct((1, N), jnp.float32),
    )(degp2d)


def _mm_tc(x_ref, w_ref, dis_ref, o_ref):
    xw = jnp.dot(x_ref[...], w_ref[...], preferred_element_type=jnp.float32)
    o_ref[...] = dis_ref[...] * xw


def _matmul_prescaled(x, w, dis, bn=2000):
    return pl.pallas_call(
        _mm_tc,
        grid=(N // bn,),
        in_specs=[
            pl.BlockSpec((bn, D), lambda i: (i, 0)),
            pl.BlockSpec((D, D), lambda i: (0, 0)),
            pl.BlockSpec((bn, 1), lambda i: (i, 0)),
        ],
        out_specs=pl.BlockSpec((bn, D), lambda i: (i, 0)),
        out_shape=jax.ShapeDtypeStruct((N, D), jnp.float32),
    )(x, w, dis)


def _post1_tc(aggp_ref, yw_ref, dis_ref, b_ref, w2_ref, o_ref):
    dis = dis_ref[...]
    h = dis * (aggp_ref[0] + aggp_ref[1] + yw_ref[...]) + b_ref[...]
    h = jnp.maximum(h, 0.0)
    o_ref[...] = dis * jnp.dot(h, w2_ref[...],
                               preferred_element_type=jnp.float32)


def _layer1_post(aggp, yw1, dis, b1, w2, bn=2000):
    return pl.pallas_call(
        _post1_tc,
        grid=(N // bn,),
        in_specs=[
            pl.BlockSpec((2, bn, D), lambda i: (0, i, 0)),
            pl.BlockSpec((bn, D), lambda i: (i, 0)),
            pl.BlockSpec((bn, 1), lambda i: (i, 0)),
            pl.BlockSpec((1, D), lambda i: (0, 0)),
            pl.BlockSpec((D, D), lambda i: (0, 0)),
        ],
        out_specs=pl.BlockSpec((bn, D), lambda i: (i, 0)),
        out_shape=jax.ShapeDtypeStruct((N, D), jnp.float32),
    )(aggp, yw1, dis, b1, w2)


def _post2_tc(aggp_ref, yw_ref, dis_ref, b_ref, h_ref, ls_ref):
    dis = dis_ref[...]
    h = dis * (aggp_ref[0] + aggp_ref[1] + yw_ref[...]) + b_ref[...]
    h_ref[...] = h
    m = jnp.max(h, axis=-1, keepdims=True)
    lse = jnp.log(jnp.sum(jnp.exp(h - m), axis=-1, keepdims=True)) + m
    ls_ref[...] = h - lse


def _layer2_post(aggp, yw2, dis, b2, bn=2000):
    return pl.pallas_call(
        _post2_tc,
        grid=(N // bn,),
        in_specs=[
            pl.BlockSpec((2, bn, D), lambda i: (0, i, 0)),
            pl.BlockSpec((bn, D), lambda i: (i, 0)),
            pl.BlockSpec((bn, 1), lambda i: (i, 0)),
            pl.BlockSpec((1, D), lambda i: (0, 0)),
        ],
        out_specs=[
            pl.BlockSpec((bn, D), lambda i: (i, 0)),
            pl.BlockSpec((bn, D), lambda i: (i, 0)),
        ],
        out_shape=[
            jax.ShapeDtypeStruct((N, D), jnp.float32),
            jax.ShapeDtypeStruct((N, D), jnp.float32),
        ],
    )(aggp, yw2, dis, b2)


_MESH = dict(core_axis_name="c", subcore_axis_name="s")


@functools.partial(
    pl.kernel,
    out_type=jax.ShapeDtypeStruct((NC, NS, 1, NPT), jnp.float32),
    mesh=plsc.VectorSubcoreMesh(**_MESH),
    scratch_types=[
        pltpu.VMEM((NCH, CH), jnp.int32),
        pltpu.VMEM((EW,), jnp.float32),
        pltpu.VMEM((NPT,), jnp.float32),
        pltpu.VMEM_SHARED((NP,), jnp.float32),
    ],
)
def _deg_kernel(col_hbm, ew_hbm, out_hbm, col_v, ew_v, z_v, degw):
    c = lax.axis_index("c")
    s = lax.axis_index("s")
    w = s * NC + c

    def zgrp(j, carry):
        z_v[pl.ds(j * L, L)] = jnp.zeros((L,), jnp.float32)
        return carry

    lax.fori_loop(0, NPT // L, zgrp, 0)
    pltpu.sync_copy(z_v, degw.at[pl.ds(s * NPT, NPT)])
    plsc.subcore_barrier()

    pltpu.sync_copy(col_hbm.at[w], col_v)
    pltpu.sync_copy(ew_hbm.at[pl.ds(w * EW, EW)], ew_v)

    def chunk(j, carry):
        pltpu.sync_copy(ew_v.at[pl.ds(j * CH, CH)],
                        degw.at[col_v.at[j]], add=True)
        return carry

    lax.fori_loop(0, NCH, chunk, 0)
    plsc.subcore_barrier()
    pltpu.sync_copy(degw.at[pl.ds(s * NPT, NPT)], out_hbm.at[c, s, 0])


NBUF = 2
CH3 = 80
EW3 = 10240
NCH3 = EW3 // CH3
NSUP = NCH3 // NBUF
EPAD = EW3 - EW
GRP3 = CH3 // L
NPT2 = N // NS


@functools.partial(
    pl.kernel,
    out_type=jax.ShapeDtypeStruct((NC, NS, NPT2, D), jnp.float32),
    mesh=plsc.VectorSubcoreMesh(**_MESH),
    scratch_types=[
        pltpu.VMEM((EW3,), jnp.int32),
        pltpu.VMEM((EW3,), jnp.float32),
        pltpu.VMEM_SHARED((N, D), jnp.float32),
    ]
    + [pltpu.VMEM((CH3,), jnp.int32) for _ in range(NBUF)]
    + [pltpu.VMEM((CH3,), jnp.int32) for _ in range(NBUF)]
    + [pltpu.VMEM((CH3, D), jnp.float32) for _ in range(NBUF)]
    + [pltpu.SemaphoreType.DMA for _ in range(2 * NBUF)],
)
def _spmm_kernel(yw_hbm, packed_hbm, ewf_hbm, aggp_hbm,
                 packed_v, ewf_v, acc, *sc):
    rowi = sc[:NBUF]
    coli = sc[NBUF:2 * NBUF]
    msgs = sc[2 * NBUF:3 * NBUF]
    gsem = sc[3 * NBUF:4 * NBUF]
    wsem = sc[4 * NBUF:]
    c = lax.axis_index("c")
    s = lax.axis_index("s")
    w = s * NC + c

    def unpack_rows(j, b):
        for g in range(GRP3):
            pv = packed_v[pl.ds(j * CH3 + g * L, L)]
            rowi[b][pl.ds(g * L, L)] = jnp.bitwise_and(pv, 0xFFFF)

    def unpack_cols(j, b):
        for g in range(GRP3):
            pv = packed_v[pl.ds(j * CH3 + g * L, L)]
            coli[b][pl.ds(g * L, L)] = jnp.right_shift(pv, 16)

    def gather_start(b):
        pltpu.make_async_copy(yw_hbm.at[rowi[b]], msgs[b], gsem[b]).start()

    def gather_wait(b):
        pltpu.make_async_copy(yw_hbm.at[rowi[b]], msgs[b], gsem[b]).wait()

    def scatter_start(b):
        pltpu.async_copy(msgs[b], acc.at[coli[b]], wsem[b], add=True)

    def scatter_wait(b):
        pltpu.make_async_copy(msgs[b], acc.at[coli[b]], wsem[b]).wait()

    def zrow(j, carry):
        for u in range(D // L):
            msgs[0][j, pl.ds(u * L, L)] = jnp.zeros((L,), jnp.float32)
        return carry

    lax.fori_loop(0, CH3, zrow, 0)
    nfull = NPT2 // CH3
    for q in range(nfull):
        pltpu.sync_copy(msgs[0], acc.at[pl.ds(s * NPT2 + q * CH3, CH3)])
    rem = NPT2 - nfull * CH3
    if rem:
        pltpu.sync_copy(msgs[0].at[pl.ds(0, rem)],
                        acc.at[pl.ds(s * NPT2 + nfull * CH3, rem)])

    pltpu.sync_copy(packed_hbm.at[pl.ds(w * EW3, EW3)], packed_v)
    pltpu.sync_copy(ewf_hbm.at[pl.ds(w * EW3, EW3)], ewf_v)
    plsc.subcore_barrier()

    unpack_rows(0, 0)
    gather_start(0)

    def super_it(t, carry):
        for b in range(NBUF):
            j = t * NBUF + b
            bn = (b + NBUF - 1) % NBUF
            gather_wait(b)

            @pl.when(j >= 1)
            def _drain_prev_scatter():
                scatter_wait(bn)

            @pl.when(j + 1 <= NCH3 - 1)
            def _gather_ahead():
                unpack_rows(j + 1, bn)
                gather_start(bn)

            unpack_cols(j, b)
            for g in range(GRP3):
                evec = ewf_v[pl.ds(j * CH3 + g * L, L)]
                for l in range(L):
                    svec = jnp.full((L,), evec[l], jnp.float32)
                    row = g * L + l
                    for u in range(D // L):
                        msgs[b][row, pl.ds(u * L, L)] = \
                            msgs[b][row, pl.ds(u * L, L)] * svec
            scatter_start(b)
        return carry

    lax.fori_loop(0, NSUP, super_it, 0)
    scatter_wait((NCH3 - 1) % NBUF)
    plsc.subcore_barrier()
    pltpu.sync_copy(acc.at[pl.ds(s * NPT2, NPT2)], aggp_hbm.at[c, s])


def kernel(x, edge_index, edge_count, W1, b1, W2, b2):
    rowf = edge_index[0].astype(jnp.int32)
    colf = edge_index[1].astype(jnp.int32)
    col3d = colf.reshape(NW, NCH, CH)
    ec2d = edge_count[:, 0].reshape(E // D, D)

    ewf = _edge_weights(ec2d).reshape(E)
    degp = _deg_kernel(col3d, ewf)
    dis = _deg_inv_sqrt(degp.reshape(NC, NP))
    dis = dis.reshape(N, 1)

    def padw(a):
        return jnp.pad(a.reshape(NW, EW), ((0, 0), (0, EPAD))).reshape(-1)

    packed = padw(jnp.left_shift(colf, 16) | rowf)
    ewp = padw(ewf)

    yw1 = _matmul_prescaled(x, W1, dis)
    aggp1 = _spmm_kernel(yw1, packed, ewp).reshape(NC, N, D)
    yw2 = _layer1_post(aggp1, yw1, dis, b1.reshape(1, D), W2)
    aggp2 = _spmm_kernel(yw2, packed, ewp).reshape(NC, N, D)
    h2, ls = _layer2_post(aggp2, yw2, dis, b2.reshape(1, D))
    return (h2, ls)


# speedup vs baseline: 1.0005x; 1.0005x over previous
import functools

import jax
import jax.numpy as jnp
from jax import lax
from jax.experimental import pallas as pl
from jax.experimental.pallas import tpu as pltpu
from jax.experimental.pallas import tpu_sc as plsc

N = 10000
E = 320000
D = 128
NC = 2
NS = 16
NW = NC * NS
L = 16
CH = 80
NCH = (E // NW) // CH
EW = E // NW
NP = 10240
NPT = NP // NS
GRP = CH // L


def _ew_tc(ec_ref, ew_ref):
    e = ec_ref[...]
    mn = jnp.min(e)
    mx = jnp.max(e)
    ew_ref[...] = (e - mn) / (mx - mn)


def _edge_weights(ec2d):
    return pl.pallas_call(
        _ew_tc,
        out_shape=jax.ShapeDtypeStruct(ec2d.shape, jnp.float32),
    )(ec2d)


def _dis_tc(degp_ref, dis_ref):
    d = degp_ref[:1, :N] + degp_ref[1:, :N] + 1.0
    dis_ref[...] = jax.lax.rsqrt(jnp.maximum(d, 1e-12))


def _deg_inv_sqrt(degp2d):
    return pl.pallas_call(
        _dis_tc,
        out_shape=jax.ShapeDtypeStruct((1, N), jnp.float32),
    )(degp2d)


def _mm_tc(x_ref, w_ref, dis_ref, o_ref):
    xw = jnp.dot(x_ref[...], w_ref[...], preferred_element_type=jnp.float32)
    o_ref[...] = dis_ref[...] * xw


def _matmul_prescaled(x, w, dis, bn=2000):
    return pl.pallas_call(
        _mm_tc,
        grid=(N // bn,),
        in_specs=[
            pl.BlockSpec((bn, D), lambda i: (i, 0)),
            pl.BlockSpec((D, D), lambda i: (0, 0)),
            pl.BlockSpec((bn, 1), lambda i: (i, 0)),
        ],
        out_specs=pl.BlockSpec((bn, D), lambda i: (i, 0)),
        out_shape=jax.ShapeDtypeStruct((N, D), jnp.float32),
    )(x, w, dis)


def _post1_tc(aggp_ref, yw_ref, dis_ref, b_ref, w2_ref, o_ref):
    dis = dis_ref[...]
    h = dis * (aggp_ref[0] + aggp_ref[1] + yw_ref[...]) + b_ref[...]
    h = jnp.maximum(h, 0.0)
    o_ref[...] = dis * jnp.dot(h, w2_ref[...],
                               preferred_element_type=jnp.float32)


def _layer1_post(aggp, yw1, dis, b1, w2, bn=2000):
    return pl.pallas_call(
        _post1_tc,
        grid=(N // bn,),
        in_specs=[
            pl.BlockSpec((2, bn, D), lambda i: (0, i, 0)),
            pl.BlockSpec((bn, D), lambda i: (i, 0)),
            pl.BlockSpec((bn, 1), lambda i: (i, 0)),
            pl.BlockSpec((1, D), lambda i: (0, 0)),
            pl.BlockSpec((D, D), lambda i: (0, 0)),
        ],
        out_specs=pl.BlockSpec((bn, D), lambda i: (i, 0)),
        out_shape=jax.ShapeDtypeStruct((N, D), jnp.float32),
    )(aggp, yw1, dis, b1, w2)


def _post2_tc(aggp_ref, yw_ref, dis_ref, b_ref, h_ref, ls_ref):
    dis = dis_ref[...]
    h = dis * (aggp_ref[0] + aggp_ref[1] + yw_ref[...]) + b_ref[...]
    h_ref[...] = h
    m = jnp.max(h, axis=-1, keepdims=True)
    lse = jnp.log(jnp.sum(jnp.exp(h - m), axis=-1, keepdims=True)) + m
    ls_ref[...] = h - lse


def _layer2_post(aggp, yw2, dis, b2, bn=2000):
    return pl.pallas_call(
        _post2_tc,
        grid=(N // bn,),
        in_specs=[
            pl.BlockSpec((2, bn, D), lambda i: (0, i, 0)),
            pl.BlockSpec((bn, D), lambda i: (i, 0)),
            pl.BlockSpec((bn, 1), lambda i: (i, 0)),
            pl.BlockSpec((1, D), lambda i: (0, 0)),
        ],
        out_specs=[
            pl.BlockSpec((bn, D), lambda i: (i, 0)),
            pl.BlockSpec((bn, D), lambda i: (i, 0)),
        ],
        out_shape=[
            jax.ShapeDtypeStruct((N, D), jnp.float32),
            jax.ShapeDtypeStruct((N, D), jnp.float32),
        ],
    )(aggp, yw2, dis, b2)


_MESH = dict(core_axis_name="c", subcore_axis_name="s")


@functools.partial(
    pl.kernel,
    out_type=jax.ShapeDtypeStruct((NC, NS, 1, NPT), jnp.float32),
    mesh=plsc.VectorSubcoreMesh(**_MESH),
    scratch_types=[
        pltpu.VMEM((NCH, CH), jnp.int32),
        pltpu.VMEM((EW,), jnp.float32),
        pltpu.VMEM((NPT,), jnp.float32),
        pltpu.VMEM_SHARED((NP,), jnp.float32),
    ],
)
def _deg_kernel(col_hbm, ew_hbm, out_hbm, col_v, ew_v, z_v, degw):
    c = lax.axis_index("c")
    s = lax.axis_index("s")
    w = s * NC + c

    def zgrp(j, carry):
        z_v[pl.ds(j * L, L)] = jnp.zeros((L,), jnp.float32)
        return carry

    lax.fori_loop(0, NPT // L, zgrp, 0)
    pltpu.sync_copy(z_v, degw.at[pl.ds(s * NPT, NPT)])
    plsc.subcore_barrier()

    pltpu.sync_copy(col_hbm.at[w], col_v)
    pltpu.sync_copy(ew_hbm.at[pl.ds(w * EW, EW)], ew_v)

    def chunk(j, carry):
        pltpu.sync_copy(ew_v.at[pl.ds(j * CH, CH)],
                        degw.at[col_v.at[j]], add=True)
        return carry

    lax.fori_loop(0, NCH, chunk, 0)
    plsc.subcore_barrier()
    pltpu.sync_copy(degw.at[pl.ds(s * NPT, NPT)], out_hbm.at[c, s, 0])


NBUF = 2
CH3 = 80
EW3 = 10240
NCH3 = EW3 // CH3
NSUP = NCH3 // NBUF
EPAD = EW3 - EW
GRP3 = CH3 // L
NPT2 = N // NS


@functools.partial(
    pl.kernel,
    out_type=jax.ShapeDtypeStruct((NC, NS, NPT2, D), jnp.float32),
    mesh=plsc.VectorSubcoreMesh(**_MESH),
    scratch_types=[
        pltpu.VMEM((EW3,), jnp.int32),
        pltpu.VMEM((EW3,), jnp.float32),
        pltpu.VMEM_SHARED((N, D), jnp.float32),
    ]
    + [pltpu.VMEM((CH3,), jnp.int32) for _ in range(NBUF)]
    + [pltpu.VMEM((CH3,), jnp.int32) for _ in range(NBUF)]
    + [pltpu.VMEM((CH3, D), jnp.float32) for _ in range(NBUF)]
    + [pltpu.SemaphoreType.DMA for _ in range(2 * NBUF)],
)
def _spmm_kernel(yw_hbm, packed_hbm, ewf_hbm, aggp_hbm,
                 packed_v, ewf_v, acc, *sc):
    rowi = sc[:NBUF]
    coli = sc[NBUF:2 * NBUF]
    msgs = sc[2 * NBUF:3 * NBUF]
    gsem = sc[3 * NBUF:4 * NBUF]
    wsem = sc[4 * NBUF:]
    c = lax.axis_index("c")
    s = lax.axis_index("s")
    w = s * NC + c

    def unpack_rows(j, b):
        for g in range(GRP3):
            pv = packed_v[pl.ds(j * CH3 + g * L, L)]
            rowi[b][pl.ds(g * L, L)] = jnp.bitwise_and(pv, 0xFFFF)

    def unpack_cols(j, b):
        for g in range(GRP3):
            pv = packed_v[pl.ds(j * CH3 + g * L, L)]
            coli[b][pl.ds(g * L, L)] = jnp.right_shift(pv, 16)

    def gather_start(b):
        pltpu.async_copy(yw_hbm.at[rowi[b]], msgs[b], gsem[b])

    def gather_wait(b):
        pltpu.make_async_copy(yw_hbm.at[pl.ds(0, CH3)], msgs[b],
                              gsem[b]).wait()

    def scatter_start(b):
        pltpu.async_copy(msgs[b], acc.at[coli[b]], wsem[b], add=True)

    def scatter_wait(b):
        pltpu.make_async_copy(msgs[b], aggp_hbm.at[c, s, pl.ds(0, CH3)],
                              wsem[b]).wait()

    def zrow(j, carry):
        for u in range(D // L):
            msgs[0][j, pl.ds(u * L, L)] = jnp.zeros((L,), jnp.float32)
        return carry

    lax.fori_loop(0, CH3, zrow, 0)
    nfull = NPT2 // CH3
    for q in range(nfull):
        pltpu.sync_copy(msgs[0], acc.at[pl.ds(s * NPT2 + q * CH3, CH3)])
    rem = NPT2 - nfull * CH3
    if rem:
        pltpu.sync_copy(msgs[0].at[pl.ds(0, rem)],
                        acc.at[pl.ds(s * NPT2 + nfull * CH3, rem)])

    pltpu.sync_copy(packed_hbm.at[pl.ds(w * EW3, EW3)], packed_v)
    pltpu.sync_copy(ewf_hbm.at[pl.ds(w * EW3, EW3)], ewf_v)
    plsc.subcore_barrier()

    unpack_rows(0, 0)
    gather_start(0)

    def super_it(t, carry):
        for b in range(NBUF):
            j = t * NBUF + b
            bn = (b + NBUF - 1) % NBUF
            gather_wait(b)

            @pl.when(j >= 1)
            def _drain_prev_scatter():
                scatter_wait(bn)

            @pl.when(j + 1 <= NCH3 - 1)
            def _gather_ahead():
                unpack_rows(j + 1, bn)
                gather_start(bn)

            unpack_cols(j, b)
            for g in range(GRP3):
                evec = ewf_v[pl.ds(j * CH3 + g * L, L)]
                for l in range(L):
                    svec = jnp.full((L,), evec[l], jnp.float32)
                    row = g * L + l
                    for u in range(D // L):
                        msgs[b][row, pl.ds(u * L, L)] = \
                            msgs[b][row, pl.ds(u * L, L)] * svec
            scatter_start(b)
        return carry

    lax.fori_loop(0, NSUP, super_it, 0)
    scatter_wait((NCH3 - 1) % NBUF)
    plsc.subcore_barrier()
    pltpu.sync_copy(acc.at[pl.ds(s * NPT2, NPT2)], aggp_hbm.at[c, s])


def kernel(x, edge_index, edge_count, W1, b1, W2, b2):
    rowf = edge_index[0].astype(jnp.int32)
    colf = edge_index[1].astype(jnp.int32)
    col3d = colf.reshape(NW, NCH, CH)
    ec2d = edge_count[:, 0].reshape(E // D, D)

    ewf = _edge_weights(ec2d).reshape(E)
    degp = _deg_kernel(col3d, ewf)
    dis = _deg_inv_sqrt(degp.reshape(NC, NP))
    dis = dis.reshape(N, 1)

    def padw(a):
        return jnp.pad(a.reshape(NW, EW), ((0, 0), (0, EPAD))).reshape(-1)

    packed = padw(jnp.left_shift(colf, 16) | rowf)
    ewp = padw(ewf)

    yw1 = _matmul_prescaled(x, W1, dis)
    aggp1 = _spmm_kernel(yw1, packed, ewp).reshape(NC, N, D)
    yw2 = _layer1_post(aggp1, yw1, dis, b1.reshape(1, D), W2)
    aggp2 = _spmm_kernel(yw2, packed, ewp).reshape(NC, N, D)
    h2, ls = _layer2_post(aggp2, yw2, dis, b2.reshape(1, D))
    return (h2, ls)


# speedup vs baseline: 1.6368x; 1.6361x over previous
import functools

import jax
import jax.numpy as jnp
from jax import lax
from jax.experimental import pallas as pl
from jax.experimental.pallas import tpu as pltpu
from jax.experimental.pallas import tpu_sc as plsc

N = 10000
E = 320000
D = 128
NC = 2
NS = 16
NW = NC * NS
L = 16
CH = 80
NCH = (E // NW) // CH
EW = E // NW
NP = 10240
NPT = NP // NS
GRP = CH // L


def _ew_tc(ec_ref, ew_ref):
    e = ec_ref[...]
    mn = jnp.min(e)
    mx = jnp.max(e)
    ew_ref[...] = (e - mn) / (mx - mn)


def _edge_weights(ec2d):
    return pl.pallas_call(
        _ew_tc,
        out_shape=jax.ShapeDtypeStruct(ec2d.shape, jnp.float32),
    )(ec2d)


def _dis_tc(degp_ref, dis_ref):
    d = degp_ref[:1, :N] + degp_ref[1:, :N] + 1.0
    dis_ref[...] = jax.lax.rsqrt(jnp.maximum(d, 1e-12))


def _deg_inv_sqrt(degp2d):
    return pl.pallas_call(
        _dis_tc,
        out_shape=jax.ShapeDtypeStruct((1, N), jnp.float32),
    )(degp2d)


def _mm_tc(x_ref, w_ref, dis_ref, o_ref):
    xw = jnp.dot(x_ref[...], w_ref[...], preferred_element_type=jnp.float32)
    o_ref[...] = dis_ref[...] * xw


def _matmul_prescaled(x, w, dis, bn=2000):
    return pl.pallas_call(
        _mm_tc,
        grid=(N // bn,),
        in_specs=[
            pl.BlockSpec((bn, D), lambda i: (i, 0)),
            pl.BlockSpec((D, D), lambda i: (0, 0)),
            pl.BlockSpec((bn, 1), lambda i: (i, 0)),
        ],
        out_specs=pl.BlockSpec((bn, D), lambda i: (i, 0)),
        out_shape=jax.ShapeDtypeStruct((N, D), jnp.float32),
    )(x, w, dis)


def _post1_tc(aggp_ref, yw_ref, dis_ref, b_ref, w2_ref, o_ref):
    dis = dis_ref[...]
    h = dis * (aggp_ref[0] + aggp_ref[1] + yw_ref[...]) + b_ref[...]
    h = jnp.maximum(h, 0.0)
    o_ref[...] = dis * jnp.dot(h, w2_ref[...],
                               preferred_element_type=jnp.float32)


def _layer1_post(aggp, yw1, dis, b1, w2, bn=2000):
    return pl.pallas_call(
        _post1_tc,
        grid=(N // bn,),
        in_specs=[
            pl.BlockSpec((2, bn, D), lambda i: (0, i, 0)),
            pl.BlockSpec((bn, D), lambda i: (i, 0)),
            pl.BlockSpec((bn, 1), lambda i: (i, 0)),
            pl.BlockSpec((1, D), lambda i: (0, 0)),
            pl.BlockSpec((D, D), lambda i: (0, 0)),
        ],
        out_specs=pl.BlockSpec((bn, D), lambda i: (i, 0)),
        out_shape=jax.ShapeDtypeStruct((N, D), jnp.float32),
    )(aggp, yw1, dis, b1, w2)


def _post2_tc(aggp_ref, yw_ref, dis_ref, b_ref, h_ref, ls_ref):
    dis = dis_ref[...]
    h = dis * (aggp_ref[0] + aggp_ref[1] + yw_ref[...]) + b_ref[...]
    h_ref[...] = h
    m = jnp.max(h, axis=-1, keepdims=True)
    lse = jnp.log(jnp.sum(jnp.exp(h - m), axis=-1, keepdims=True)) + m
    ls_ref[...] = h - lse


def _layer2_post(aggp, yw2, dis, b2, bn=2000):
    return pl.pallas_call(
        _post2_tc,
        grid=(N // bn,),
        in_specs=[
            pl.BlockSpec((2, bn, D), lambda i: (0, i, 0)),
            pl.BlockSpec((bn, D), lambda i: (i, 0)),
            pl.BlockSpec((bn, 1), lambda i: (i, 0)),
            pl.BlockSpec((1, D), lambda i: (0, 0)),
        ],
        out_specs=[
            pl.BlockSpec((bn, D), lambda i: (i, 0)),
            pl.BlockSpec((bn, D), lambda i: (i, 0)),
        ],
        out_shape=[
            jax.ShapeDtypeStruct((N, D), jnp.float32),
            jax.ShapeDtypeStruct((N, D), jnp.float32),
        ],
    )(aggp, yw2, dis, b2)


_MESH = dict(core_axis_name="c", subcore_axis_name="s")


@functools.partial(
    pl.kernel,
    out_type=jax.ShapeDtypeStruct((NC, NS, 1, NPT), jnp.float32),
    mesh=plsc.VectorSubcoreMesh(**_MESH),
    scratch_types=[
        pltpu.VMEM((NCH, CH), jnp.int32),
        pltpu.VMEM((EW,), jnp.float32),
        pltpu.VMEM((NPT,), jnp.float32),
        pltpu.VMEM_SHARED((NP,), jnp.float32),
    ],
)
def _deg_kernel(col_hbm, ew_hbm, out_hbm, col_v, ew_v, z_v, degw):
    c = lax.axis_index("c")
    s = lax.axis_index("s")
    w = s * NC + c

    def zgrp(j, carry):
        z_v[pl.ds(j * L, L)] = jnp.zeros((L,), jnp.float32)
        return carry

    lax.fori_loop(0, NPT // L, zgrp, 0)
    pltpu.sync_copy(z_v, degw.at[pl.ds(s * NPT, NPT)])
    plsc.subcore_barrier()

    pltpu.sync_copy(col_hbm.at[w], col_v)
    pltpu.sync_copy(ew_hbm.at[pl.ds(w * EW, EW)], ew_v)

    def chunk(j, carry):
        pltpu.sync_copy(ew_v.at[pl.ds(j * CH, CH)],
                        degw.at[col_v.at[j]], add=True)
        return carry

    lax.fori_loop(0, NCH, chunk, 0)
    plsc.subcore_barrier()
    pltpu.sync_copy(degw.at[pl.ds(s * NPT, NPT)], out_hbm.at[c, s, 0])


@functools.partial(
    pl.kernel,
    out_type=jax.ShapeDtypeStruct((NC, NP, D), jnp.float32),
    mesh=plsc.VectorSubcoreMesh(**_MESH),
    scratch_types=[
        pltpu.VMEM((EW,), jnp.int32),
        pltpu.VMEM((NCH, CH), jnp.int32),
        pltpu.VMEM((EW,), jnp.float32),
        pltpu.VMEM((CH, D), jnp.float32),
        pltpu.VMEM_SHARED((NP, D), jnp.float32),
        pltpu.SemaphoreType.DMA,
    ],
)
def _spmm_kernel(yw_hbm, rowf_hbm, col_hbm, ewf_hbm, aggp_hbm,
                 rowf_v, col_v, ewf_v, msgs_v, acc, sem):
    c = lax.axis_index("c")
    s = lax.axis_index("s")
    w = s * NC + c

    def zrow(j, carry):
        for u in range(D // L):
            msgs_v[j, pl.ds(u * L, L)] = jnp.zeros((L,), jnp.float32)
        return carry

    lax.fori_loop(0, CH, zrow, 0)
    for q in range(NPT // CH):
        pltpu.sync_copy(msgs_v, acc.at[pl.ds(s * NPT + q * CH, CH)])

    pltpu.sync_copy(rowf_hbm.at[pl.ds(w * EW, EW)], rowf_v)
    pltpu.sync_copy(col_hbm.at[w], col_v)
    pltpu.sync_copy(ewf_hbm.at[pl.ds(w * EW, EW)], ewf_v)
    plsc.subcore_barrier()

    def chunk(j, carry):
        pltpu.async_copy(
            yw_hbm.at[rowf_v.at[pl.ds(j * CH, CH)]], msgs_v, sem).wait()
        for g in range(GRP):
            evec = ewf_v[pl.ds(j * CH + g * L, L)]
            for l in range(L):
                svec = jnp.full((L,), evec[l], jnp.float32)
                for u in range(D // L):
                    msgs_v[g * L + l, pl.ds(u * L, L)] = \
                        msgs_v[g * L + l, pl.ds(u * L, L)] * svec
        pltpu.sync_copy(msgs_v, acc.at[col_v.at[j]], add=True)
        return carry

    lax.fori_loop(0, NCH, chunk, 0)
    plsc.subcore_barrier()
    pltpu.sync_copy(acc.at[pl.ds(s * NPT, NPT)],
                    aggp_hbm.at[c, pl.ds(s * NPT, NPT)])


def kernel(x, edge_index, edge_count, W1, b1, W2, b2):
    rowf = edge_index[0].astype(jnp.int32)
    colf = edge_index[1].astype(jnp.int32)
    col3d = colf.reshape(NW, NCH, CH)
    ec2d = edge_count[:, 0].reshape(E // D, D)

    ewf = _edge_weights(ec2d).reshape(E)
    degp = _deg_kernel(col3d, ewf)
    dis = _deg_inv_sqrt(degp.reshape(NC, NP))
    dis = dis.reshape(N, 1)

    yw1 = _matmul_prescaled(x, W1, dis)
    aggp1 = _spmm_kernel(yw1, rowf, col3d, ewf)
    yw2 = _layer1_post(aggp1, yw1, dis, b1.reshape(1, D), W2)
    aggp2 = _spmm_kernel(yw2, rowf, col3d, ewf)
    h2, ls = _layer2_post(aggp2, yw2, dis, b2.reshape(1, D))
    return (h2, ls)
